# Initial kernel scaffold; baseline (speedup 1.0000x reference)
import jax
import jax.numpy as jnp
from jax.experimental import pallas as pl


def kernel(depth_pred, depth_gt, mask_gt, xy1_grid, hsv):
    raise NotImplementedError("write your pallas kernel here")


# speedup vs baseline: 2.5404x; 2.5404x over previous
import functools

import jax
import jax.numpy as jnp
from jax.experimental import pallas as pl
from jax.experimental.pallas import tpu as pltpu

_R = 2
_INV2SX = 1.0 / (2.0 * 0.05 ** 2)
_INV2SH = 1.0 / (2.0 * 0.1 ** 2)
_PAD = 1e4


def _stencil_body(xy1_ref, dgt_ref, msk_ref, hsv_ref,
                  xy1p_ref, dpp_ref, hsvp_ref,
                  psum_ref, cnt_ref):
    H, W = dgt_ref.shape[1], dgt_ref.shape[2]
    xy1 = xy1_ref[0]
    dgt = dgt_ref[0]
    hsv = hsv_ref[0]
    xyzg = xy1 * dgt[None]
    xy1p = xy1p_ref[0]
    dpp = dpp_ref[0]
    xyzp = xy1p * dpp[None]
    hsvp = hsvp_ref[0]
    total = jnp.zeros((H, W), dtype=jnp.float32)
    for dy in range(2 * _R + 1):
        for dx in range(2 * _R + 1):
            xs = xyzp[:, dy:dy + H, dx:dx + W]
            hs = hsvp[:, dy:dy + H, dx:dx + W]
            d2 = jnp.sum((xyzg - xs) ** 2, axis=0)
            h2 = jnp.sum((hsv - hs) ** 2, axis=0)
            total = total + jnp.exp(-(d2 * _INV2SX + h2 * _INV2SH))
    msk = msk_ref[0]
    psum_ref[0, 0, :] = jnp.full((128,), jnp.sum(total * msk), jnp.float32)
    cnt_ref[0, 0, :] = jnp.full((128,), jnp.sum(msk), jnp.float32)


def kernel(depth_pred, depth_gt, mask_gt, xy1_grid, hsv):
    B, _, H, W = depth_pred.shape
    r = _R
    dgt = depth_gt[:, 0]
    msk = mask_gt[:, 0].astype(jnp.float32)
    padhw = ((0, 0), (r, r), (r, r))
    pad3 = ((0, 0), (0, 0), (r, r), (r, r))
    dpp = jnp.pad(depth_pred[:, 0], padhw, constant_values=_PAD)
    xy1p = jnp.pad(xy1_grid, pad3, constant_values=1.0)
    hsvp = jnp.pad(hsv, pad3, constant_values=_PAD)

    Hp, Wp = H + 2 * r, W + 2 * r
    grid = (B,)
    b3 = lambda b: (b, 0, 0, 0)
    b2 = lambda b: (b, 0, 0)
    psum, cnt = pl.pallas_call(
        _stencil_body,
        grid=grid,
        in_specs=[
            pl.BlockSpec((1, 3, H, W), b3),
            pl.BlockSpec((1, H, W), b2),
            pl.BlockSpec((1, H, W), b2),
            pl.BlockSpec((1, 3, H, W), b3),
            pl.BlockSpec((1, 3, Hp, Wp), b3),
            pl.BlockSpec((1, Hp, Wp), b2),
            pl.BlockSpec((1, 3, Hp, Wp), b3),
        ],
        out_specs=[
            pl.BlockSpec((1, 1, 128), lambda b: (b, 0, 0)),
            pl.BlockSpec((1, 1, 128), lambda b: (b, 0, 0)),
        ],
        out_shape=[
            jax.ShapeDtypeStruct((B, 1, 128), jnp.float32),
            jax.ShapeDtypeStruct((B, 1, 128), jnp.float32),
        ],
    )(xy1_grid, dgt, msk, hsv, xy1p, dpp, hsvp)
    psum = psum[:, 0, 0]
    cnt = cnt[:, 0, 0]
    n_valid = jnp.sum(cnt)
    inp = jnp.sum(psum) / (n_valid * float((2 * r + 1) ** 2) + 1e-8)
    return 1.0 - inp
